# Initial kernel scaffold; baseline (speedup 1.0000x reference)
import jax
import jax.numpy as jnp
from jax.experimental import pallas as pl


def kernel(ids, u, b):
    raise NotImplementedError("write your pallas kernel here")


# speedup vs baseline: 5.9821x; 5.9821x over previous
import functools

import jax
import jax.numpy as jnp
from jax import lax
from jax.experimental import pallas as pl
from jax.experimental.pallas import tpu as pltpu
from jax.experimental.pallas import tpu_sc as plsc

VOCAB = 100000
HD = 64
HASH = 4096
B = 4096
L = 50
N = B * L
NC = 2
NS = 16
NW = NC * NS
CHUNK = N // NW
GB = 128
NG = CHUNK // GB
MULT = VOCAB % HASH
MASK = HASH - 1

_mesh = plsc.VectorSubcoreMesh(core_axis_name="c", subcore_axis_name="s")


@functools.partial(
    pl.kernel,
    out_type=jax.ShapeDtypeStruct((N, 2 * HD), jnp.float32),
    mesh=_mesh,
    compiler_params=pltpu.CompilerParams(use_tc_tiling_on_sc=False),
    scratch_types=[
        pltpu.VMEM((CHUNK + 16,), jnp.int32),
        pltpu.VMEM((CHUNK,), jnp.int32),
        pltpu.VMEM((GB, HD), jnp.float32),
        pltpu.VMEM((GB, HD), jnp.float32),
        pltpu.SemaphoreType.DMA,
        pltpu.SemaphoreType.DMA,
    ],
)
def _bigram_gather(ids_hbm, u_hbm, b_hbm, out_hbm,
                   ids_v, bi_v, ue_v, be_v, sem_u, sem_b):
    wid = lax.axis_index("s") * NC + lax.axis_index("c")
    base = wid * CHUNK
    pltpu.sync_copy(ids_hbm.at[pl.ds(base, CHUNK)], ids_v.at[pl.ds(16, CHUNK)])

    lanes = lax.iota(jnp.int32, 16)

    def bi_body(j, carry):
        o = 16 + j * 16
        cur = ids_v[pl.ds(o, 16)]
        prev = ids_v[pl.ds(o - 1, 16)]
        pos = (j * 16 + lanes) % L
        pi = jnp.where(pos == 0, 0, prev)
        bi_v[pl.ds(j * 16, 16)] = ((pi & MASK) * MULT + cur) & MASK
        return carry

    lax.fori_loop(0, CHUNK // 16, bi_body, 0)

    def g_body(g, carry):
        ro = base + g * GB
        cu = pltpu.async_copy(
            u_hbm.at[ids_v.at[pl.ds(16 + g * GB, GB)]], ue_v, sem_u)
        cb = pltpu.async_copy(
            b_hbm.at[bi_v.at[pl.ds(g * GB, GB)]], be_v, sem_b)
        cu.wait()
        cb.wait()
        pltpu.sync_copy(ue_v, out_hbm.at[pl.ds(ro, GB), pl.ds(0, HD)])
        pltpu.sync_copy(be_v, out_hbm.at[pl.ds(ro, GB), pl.ds(HD, HD)])
        return carry

    lax.fori_loop(0, NG, g_body, 0)


def kernel(ids, u, b):
    out = _bigram_gather(ids.reshape(N), u, b)
    return out.reshape(B, L, 2 * HD)


# speedup vs baseline: 6.3700x; 1.0648x over previous
import functools

import jax
import jax.numpy as jnp
from jax import lax
from jax.experimental import pallas as pl
from jax.experimental.pallas import tpu as pltpu
from jax.experimental.pallas import tpu_sc as plsc

VOCAB = 100000
HD = 64
HASH = 4096
B = 4096
L = 50
N = B * L
NC = 2
NS = 16
NW = NC * NS
CHUNK = N // NW
GB = 128
NG = CHUNK // GB
MULT = VOCAB % HASH
MASK = HASH - 1

_mesh = plsc.VectorSubcoreMesh(core_axis_name="c", subcore_axis_name="s")


@functools.partial(
    pl.kernel,
    out_type=jax.ShapeDtypeStruct((N, 2 * HD), jnp.float32),
    mesh=_mesh,
    compiler_params=pltpu.CompilerParams(use_tc_tiling_on_sc=False),
    scratch_types=[
        pltpu.VMEM((CHUNK + 16,), jnp.int32),
        pltpu.VMEM((CHUNK,), jnp.int32),
        pltpu.VMEM((2, GB, HD), jnp.float32),
        pltpu.VMEM((2, GB, HD), jnp.float32),
        pltpu.SemaphoreType.DMA((2,)),
        pltpu.SemaphoreType.DMA((2,)),
    ],
)
def _bigram_gather(ids_hbm, u_hbm, b_hbm, out_hbm,
                   ids_v, bi_v, ue_v, be_v, sem_u, sem_b):
    wid = lax.axis_index("s") * NC + lax.axis_index("c")
    base = wid * CHUNK
    pltpu.sync_copy(ids_hbm.at[pl.ds(base, CHUNK)], ids_v.at[pl.ds(16, CHUNK)])

    lanes = lax.iota(jnp.int32, 16)

    def bi_body(j, carry):
        o = 16 + j * 16
        cur = ids_v[pl.ds(o, 16)]
        prev = ids_v[pl.ds(o - 1, 16)]
        pos = (j * 16 + lanes) % L
        pi = jnp.where(pos == 0, 0, prev)
        bi_v[pl.ds(j * 16, 16)] = ((pi & MASK) * MULT + cur) & MASK
        return carry

    lax.fori_loop(0, CHUNK // 16, bi_body, 0)

    def issue(g, p):
        pltpu.async_copy(u_hbm.at[ids_v.at[pl.ds(16 + g * GB, GB)]],
                         ue_v.at[p], sem_u.at[p])
        pltpu.async_copy(b_hbm.at[bi_v.at[pl.ds(g * GB, GB)]],
                         be_v.at[p], sem_b.at[p])

    def wait_gathers(p):
        pltpu.make_async_copy(u_hbm.at[pl.ds(0, GB)], ue_v.at[p],
                              sem_u.at[p]).wait()
        pltpu.make_async_copy(b_hbm.at[pl.ds(0, GB)], be_v.at[p],
                              sem_b.at[p]).wait()

    issue(0, 0)

    def g_body(i, carry):
        for (off, p, q) in ((0, 0, 1), (1, 1, 0)):
            g = 2 * i + off

            @pl.when(g + 1 < NG)
            def _():
                issue(g + 1, q)

            wait_gathers(p)
            ro = base + g * GB
            pltpu.sync_copy(ue_v.at[p], out_hbm.at[pl.ds(ro, GB), pl.ds(0, HD)])
            pltpu.sync_copy(be_v.at[p], out_hbm.at[pl.ds(ro, GB), pl.ds(HD, HD)])
        return carry

    lax.fori_loop(0, NG // 2, g_body, 0)


def kernel(ids, u, b):
    out = _bigram_gather(ids.reshape(N), u, b)
    return out.reshape(B, L, 2 * HD)
